# Initial kernel scaffold; baseline (speedup 1.0000x reference)
import jax
import jax.numpy as jnp
from jax.experimental import pallas as pl


def kernel(x, W1, g1, b1, W2, g2, b2, W3, g3, b3, W4, g4, b4, W5, g5, b5, W6, g6, b6, W8, g8, b8, W9, g9, b9, W10, g10, b10, W11):
    raise NotImplementedError("write your pallas kernel here")


# speedup vs baseline: 5.1623x; 5.1623x over previous
import functools

import jax
import jax.numpy as jnp
from jax import lax
from jax.experimental import pallas as pl
from jax.experimental.pallas import tpu as pltpu
from jax.experimental.pallas import tpu_sc as plsc

KNN_K = 20
KPAD = 32
EPS = 1e-5
INTERPRET = False


def _knn_topk_body(xt_ref, idx_ref_write, b, i, RB, N, C):
    xt_all = xt_ref[0]
    xb = xt_ref[0, pl.ds(i * RB, RB), :]
    gram = lax.dot_general(xb, xt_all, (((1,), (1,)), ((), ())),
                           preferred_element_type=jnp.float32)
    ones = jnp.ones((1, C), jnp.float32)
    sq_row = lax.dot_general(ones, xt_all * xt_all, (((1,), (1,)), ((), ())),
                             preferred_element_type=jnp.float32,
                             precision=lax.Precision.HIGHEST)
    sqb = lax.dot_general(xb * xb, jnp.ones((C, 1), jnp.float32),
                          (((1,), (0,)), ((), ())),
                          preferred_element_type=jnp.float32,
                          precision=lax.Precision.HIGHEST)
    pd = 2.0 * gram - sqb - sq_row
    iota = lax.broadcasted_iota(jnp.int32, (RB, N), 1)
    kio = lax.broadcasted_iota(jnp.int32, (RB, KPAD), 1)

    def step(t, carry):
        pdc, acc = carry
        m = jnp.max(pdc, axis=1, keepdims=True)
        arg = jnp.min(jnp.where(pdc == m, iota, N), axis=1, keepdims=True)
        acc = jnp.where(kio == t, arg, acc)
        pdc = jnp.where(iota == arg, -jnp.inf, pdc)
        return pdc, acc

    _, acc = lax.fori_loop(0, KNN_K, step,
                           (pd, jnp.zeros((RB, KPAD), jnp.int32)))
    idx_ref_write(acc + b * N)


def _knn_idx(xt):
    B, N, C = xt.shape
    RB = 256

    def body(xt_ref, idx_ref):
        b = pl.program_id(0)
        i = pl.program_id(1)

        def wr(val):
            idx_ref[0] = val

        _knn_topk_body(xt_ref, wr, b, i, RB, N, C)

    return pl.pallas_call(
        body,
        grid=(B, N // RB),
        in_specs=[pl.BlockSpec((1, N, C), lambda b, i: (b, 0, 0))],
        out_specs=pl.BlockSpec((1, RB, KPAD), lambda b, i: (b, i, 0)),
        out_shape=jax.ShapeDtypeStruct((B, N, KPAD), jnp.int32),
        interpret=INTERPRET,
    )(xt)


def _knn_uv(xt, wu, wv):
    B, N, C = xt.shape
    Du = wu.shape[1]
    Do = wv.shape[1]
    RB = 256

    def body(xt_ref, wu_ref, wv_ref, idx_ref, u_ref, v_ref):
        b = pl.program_id(0)
        i = pl.program_id(1)

        def wr(val):
            idx_ref[0] = val

        _knn_topk_body(xt_ref, wr, b, i, RB, N, C)
        xb = xt_ref[0, pl.ds(i * RB, RB), :]
        u_ref[0] = jnp.dot(xb, wu_ref[...], preferred_element_type=jnp.float32)
        v_ref[0] = jnp.dot(xb, wv_ref[...], preferred_element_type=jnp.float32)

    return pl.pallas_call(
        body,
        grid=(B, N // RB),
        in_specs=[pl.BlockSpec((1, N, C), lambda b, i: (b, 0, 0)),
                  pl.BlockSpec((C, Du), lambda b, i: (0, 0)),
                  pl.BlockSpec((C, Do), lambda b, i: (0, 0))],
        out_specs=[pl.BlockSpec((1, RB, KPAD), lambda b, i: (b, i, 0)),
                   pl.BlockSpec((1, RB, Du), lambda b, i: (b, i, 0)),
                   pl.BlockSpec((1, RB, Do), lambda b, i: (b, i, 0))],
        out_shape=[jax.ShapeDtypeStruct((B, N, KPAD), jnp.int32),
                   jax.ShapeDtypeStruct((B, N, Du), jnp.float32),
                   jax.ShapeDtypeStruct((B, N, Do), jnp.float32)],
        interpret=INTERPRET,
    )(xt, wu, wv)


def _sc_gather(table, flat_idx):
    M = flat_idx.shape[0]
    D = table.shape[1]
    NW = 32
    PW = M // NW
    CH = 512
    nch = PW // CH
    mesh = plsc.VectorSubcoreMesh(core_axis_name="c", subcore_axis_name="s")

    @functools.partial(
        pl.kernel, mesh=mesh,
        out_type=jax.ShapeDtypeStruct((M, D), jnp.float32),
        scratch_types=[pltpu.VMEM((PW,), jnp.int32),
                       pltpu.VMEM((CH, D), jnp.float32),
                       pltpu.SemaphoreType.DMA])
    def gk(table_hbm, idx_hbm, out_hbm, idx_v, buf, sem):
        wid = lax.axis_index("s") * 2 + lax.axis_index("c")
        base = wid * PW
        pltpu.sync_copy(idx_hbm.at[pl.ds(base, PW)], idx_v)

        @pl.loop(0, nch)
        def _(c):
            pltpu.async_copy(table_hbm.at[idx_v.at[pl.ds(c * CH, CH)]],
                             buf, sem).wait()
            pltpu.sync_copy(buf, out_hbm.at[pl.ds(base + c * CH, CH)])

    return gk(table, flat_idx)


def _kahan_add(st_ref, row, p):
    s_old = st_ref[row:row + 1, :]
    c_old = st_ref[row + 1:row + 2, :]
    t = s_old + p
    st_ref[row:row + 1, :] = t
    st_ref[row + 1:row + 2, :] = c_old + (p - (t - s_old))


def _edge_stats_raw(g3, xf, wf, Cs):
    Kd, Mp, Dg = g3.shape
    Do = wf.shape[1]
    PB = 512
    NB = Mp // PB
    cnt = float(Mp * Kd)

    def body(g_ref, x_ref, w_ref, o_ref, st_ref):
        ph = pl.program_id(0)
        i = pl.program_id(1)

        @pl.when(jnp.logical_and(ph == 0, i == 0))
        def _():
            st_ref[...] = jnp.zeros_like(st_ref)

        xn = x_ref[...][:, :Cs]
        w = w_ref[...]

        @pl.when(ph == 0)
        def _():
            for k in range(Kd):
                feat = jnp.concatenate([g_ref[k][:, :Cs] - xn, xn], axis=1)
                y = jnp.dot(feat, w, preferred_element_type=jnp.float32)
                _kahan_add(st_ref, 0, jnp.sum(y, axis=0, keepdims=True))

        @pl.when(jnp.logical_and(ph == 1, i == 0))
        def _():
            st_ref[2:3, :] = (st_ref[0:1, :] + st_ref[1:2, :]) / cnt

        @pl.when(ph == 1)
        def _():
            m = st_ref[2:3, :]
            for k in range(Kd):
                feat = jnp.concatenate([g_ref[k][:, :Cs] - xn, xn], axis=1)
                y = jnp.dot(feat, w, preferred_element_type=jnp.float32)
                e = y - m
                _kahan_add(st_ref, 3, jnp.sum(e * e, axis=0, keepdims=True))

        @pl.when(jnp.logical_and(ph == 1, i == NB - 1))
        def _():
            o_ref[0:1, :] = st_ref[2:3, :]
            o_ref[1:2, :] = (st_ref[3:4, :] + st_ref[4:5, :]) / cnt

    return pl.pallas_call(
        body, grid=(2, NB),
        in_specs=[pl.BlockSpec((Kd, PB, Dg), lambda p, i: (0, i, 0)),
                  pl.BlockSpec((PB, Dg), lambda p, i: (i, 0)),
                  pl.BlockSpec((2 * Cs, Do), lambda p, i: (0, 0))],
        out_specs=pl.BlockSpec((2, Do), lambda p, i: (0, 0)),
        out_shape=jax.ShapeDtypeStruct((2, Do), jnp.float32),
        scratch_shapes=[pltpu.VMEM((8, Do), jnp.float32)],
        interpret=INTERPRET)(g3, xf, wf)


def _y_stats(y3):
    Kd, Mp, Do = y3.shape
    PB = 512
    NB = Mp // PB
    cnt = float(Mp * Kd)

    def body(y_ref, o_ref, st_ref):
        ph = pl.program_id(0)
        i = pl.program_id(1)

        @pl.when(jnp.logical_and(ph == 0, i == 0))
        def _():
            st_ref[...] = jnp.zeros_like(st_ref)

        @pl.when(ph == 0)
        def _():
            for k in range(Kd):
                _kahan_add(st_ref, 0, jnp.sum(y_ref[k], axis=0, keepdims=True))

        @pl.when(jnp.logical_and(ph == 1, i == 0))
        def _():
            st_ref[2:3, :] = (st_ref[0:1, :] + st_ref[1:2, :]) / cnt

        @pl.when(ph == 1)
        def _():
            m = st_ref[2:3, :]
            for k in range(Kd):
                e = y_ref[k] - m
                _kahan_add(st_ref, 3, jnp.sum(e * e, axis=0, keepdims=True))

        @pl.when(jnp.logical_and(ph == 1, i == NB - 1))
        def _():
            o_ref[0:1, :] = st_ref[2:3, :]
            o_ref[1:2, :] = (st_ref[3:4, :] + st_ref[4:5, :]) / cnt

    return pl.pallas_call(
        body, grid=(2, NB),
        in_specs=[pl.BlockSpec((Kd, PB, Do), lambda p, i: (0, i, 0))],
        out_specs=pl.BlockSpec((2, Do), lambda p, i: (0, 0)),
        out_shape=jax.ShapeDtypeStruct((2, Do), jnp.float32),
        scratch_shapes=[pltpu.VMEM((8, Do), jnp.float32)],
        interpret=INTERPRET)(y3)


def _edge_conv_raw(g3, xf, wf, Cs, bn1, w2t):
    Kd, Mp, Dg = g3.shape
    Dm = wf.shape[1]
    Do = w2t.shape[1]
    PB = 512

    def body(g_ref, x_ref, w_ref, bn_ref, w2_ref, y_ref, s_ref):
        i = pl.program_id(0)

        @pl.when(i == 0)
        def _():
            s_ref[...] = jnp.zeros_like(s_ref)

        xn = x_ref[...][:, :Cs]
        w = w_ref[...]
        m = bn_ref[0:1, :]
        den = jnp.sqrt(bn_ref[1:2, :] + EPS)
        gg = bn_ref[2:3, :]
        bb = bn_ref[3:4, :]
        w2 = w2_ref[...]
        s = jnp.zeros((1, Do), jnp.float32)
        q = jnp.zeros((1, Do), jnp.float32)
        for k in range(Kd):
            feat = jnp.concatenate([g_ref[k][:, :Cs] - xn, xn], axis=1)
            y1 = jnp.dot(feat, w, preferred_element_type=jnp.float32)
            p = (y1 - m) / den * gg + bb
            z = jnp.where(p >= 0, p, 0.2 * p)
            y = jnp.dot(z, w2, preferred_element_type=jnp.float32)
            y_ref[k] = y
            s = s + jnp.sum(y, axis=0, keepdims=True)
            q = q + jnp.sum(y * y, axis=0, keepdims=True)
        s_ref[0:1, :] += s
        s_ref[1:2, :] += q

    return pl.pallas_call(
        body, grid=(Mp // PB,),
        in_specs=[pl.BlockSpec((Kd, PB, Dg), lambda i: (0, i, 0)),
                  pl.BlockSpec((PB, Dg), lambda i: (i, 0)),
                  pl.BlockSpec((2 * Cs, Dm), lambda i: (0, 0)),
                  pl.BlockSpec((4, Dm), lambda i: (0, 0)),
                  pl.BlockSpec((Dm, Do), lambda i: (0, 0))],
        out_specs=[pl.BlockSpec((Kd, PB, Do), lambda i: (0, i, 0)),
                   pl.BlockSpec((2, Do), lambda i: (0, 0))],
        out_shape=[jax.ShapeDtypeStruct((Kd, Mp, Do), jnp.float32),
                   jax.ShapeDtypeStruct((2, Do), jnp.float32)],
        interpret=INTERPRET)(g3, xf, wf, bn1, w2t)


def _edge_stats(g3, vf):
    Kd, Mp, Dg = g3.shape
    D = vf.shape[1]
    PB = 512

    def body(g_ref, v_ref, s_ref):
        i = pl.program_id(0)

        @pl.when(i == 0)
        def _():
            s_ref[...] = jnp.zeros_like(s_ref)

        v = v_ref[...]
        s = jnp.zeros((1, D), jnp.float32)
        q = jnp.zeros((1, D), jnp.float32)
        for k in range(Kd):
            y = g_ref[k][:, :D] + v
            s = s + jnp.sum(y, axis=0, keepdims=True)
            q = q + jnp.sum(y * y, axis=0, keepdims=True)
        s_ref[0:1, :] += s
        s_ref[1:2, :] += q

    return pl.pallas_call(
        body, grid=(Mp // PB,),
        in_specs=[pl.BlockSpec((Kd, PB, Dg), lambda i: (0, i, 0)),
                  pl.BlockSpec((PB, D), lambda i: (i, 0))],
        out_specs=pl.BlockSpec((2, D), lambda i: (0, 0)),
        out_shape=jax.ShapeDtypeStruct((2, D), jnp.float32),
        interpret=INTERPRET)(g3, vf)


def _edge_conv(g3, vf, sc1, bi1, w2t):
    Kd, Mp, Dg = g3.shape
    D = vf.shape[1]
    Do = w2t.shape[1]
    PB = 512

    def body(g_ref, v_ref, sc_ref, bi_ref, w_ref, y_ref, s_ref):
        i = pl.program_id(0)

        @pl.when(i == 0)
        def _():
            s_ref[...] = jnp.zeros_like(s_ref)

        v = v_ref[...]
        sc = sc_ref[...]
        bi = bi_ref[...]
        w = w_ref[...]
        s = jnp.zeros((1, Do), jnp.float32)
        q = jnp.zeros((1, Do), jnp.float32)
        for k in range(Kd):
            p = (g_ref[k][:, :D] + v) * sc + bi
            z = jnp.where(p >= 0, p, 0.2 * p)
            y = jnp.dot(z, w, preferred_element_type=jnp.float32)
            y_ref[k] = y
            s = s + jnp.sum(y, axis=0, keepdims=True)
            q = q + jnp.sum(y * y, axis=0, keepdims=True)
        s_ref[0:1, :] += s
        s_ref[1:2, :] += q

    return pl.pallas_call(
        body, grid=(Mp // PB,),
        in_specs=[pl.BlockSpec((Kd, PB, Dg), lambda i: (0, i, 0)),
                  pl.BlockSpec((PB, D), lambda i: (i, 0)),
                  pl.BlockSpec((1, D), lambda i: (0, 0)),
                  pl.BlockSpec((1, D), lambda i: (0, 0)),
                  pl.BlockSpec((D, Do), lambda i: (0, 0))],
        out_specs=[pl.BlockSpec((Kd, PB, Do), lambda i: (0, i, 0)),
                   pl.BlockSpec((2, Do), lambda i: (0, 0))],
        out_shape=[jax.ShapeDtypeStruct((Kd, Mp, Do), jnp.float32),
                   jax.ShapeDtypeStruct((2, Do), jnp.float32)],
        interpret=INTERPRET)(g3, vf, sc1, bi1, w2t)


def _edge_max_y(y3, bn2):
    Kd, Mp, D = y3.shape
    PB = 512

    def body(y_ref, bn_ref, o_ref):
        m = bn_ref[0:1, :]
        den = jnp.sqrt(bn_ref[1:2, :] + EPS)
        gg = bn_ref[2:3, :]
        bb = bn_ref[3:4, :]
        acc = jnp.full((PB, D), -jnp.inf, jnp.float32)
        for k in range(Kd):
            p = (y_ref[k] - m) / den * gg + bb
            z = jnp.where(p >= 0, p, 0.2 * p)
            acc = jnp.maximum(acc, z)
        o_ref[...] = acc

    return pl.pallas_call(
        body, grid=(Mp // PB,),
        in_specs=[pl.BlockSpec((Kd, PB, D), lambda i: (0, i, 0)),
                  pl.BlockSpec((4, D), lambda i: (0, 0))],
        out_specs=pl.BlockSpec((PB, D), lambda i: (i, 0)),
        out_shape=jax.ShapeDtypeStruct((Mp, D), jnp.float32),
        interpret=INTERPRET)(y3, bn2)


def _edge_max_gv(g3, vf, sc1, bi1):
    Kd, Mp, Dg = g3.shape
    D = vf.shape[1]
    PB = 512

    def body(g_ref, v_ref, sc_ref, bi_ref, o_ref):
        v = v_ref[...]
        sc = sc_ref[...]
        bi = bi_ref[...]
        acc = jnp.full((PB, D), -jnp.inf, jnp.float32)
        for k in range(Kd):
            p = (g_ref[k][:, :D] + v) * sc + bi
            z = jnp.where(p >= 0, p, 0.2 * p)
            acc = jnp.maximum(acc, z)
        o_ref[...] = acc

    return pl.pallas_call(
        body, grid=(Mp // PB,),
        in_specs=[pl.BlockSpec((Kd, PB, Dg), lambda i: (0, i, 0)),
                  pl.BlockSpec((PB, D), lambda i: (i, 0)),
                  pl.BlockSpec((1, D), lambda i: (0, 0)),
                  pl.BlockSpec((1, D), lambda i: (0, 0))],
        out_specs=pl.BlockSpec((PB, D), lambda i: (i, 0)),
        out_shape=jax.ShapeDtypeStruct((Mp, D), jnp.float32),
        interpret=INTERPRET)(g3, vf, sc1, bi1)


def _mlp6(x1, x2, x3, wa, wb, wc):
    Mp, D = x1.shape
    Do = wa.shape[1]
    PB = 512

    def body(a_ref, b_ref, c_ref, wa_ref, wb_ref, wc_ref, y_ref, s_ref):
        i = pl.program_id(0)

        @pl.when(i == 0)
        def _():
            s_ref[...] = jnp.zeros_like(s_ref)

        y = (jnp.dot(a_ref[...], wa_ref[...], preferred_element_type=jnp.float32)
             + jnp.dot(b_ref[...], wb_ref[...], preferred_element_type=jnp.float32)
             + jnp.dot(c_ref[...], wc_ref[...], preferred_element_type=jnp.float32))
        y_ref[...] = y
        s_ref[0:1, :] += jnp.sum(y, axis=0, keepdims=True)
        s_ref[1:2, :] += jnp.sum(y * y, axis=0, keepdims=True)

    return pl.pallas_call(
        body, grid=(Mp // PB,),
        in_specs=[pl.BlockSpec((PB, D), lambda i: (i, 0)),
                  pl.BlockSpec((PB, D), lambda i: (i, 0)),
                  pl.BlockSpec((PB, D), lambda i: (i, 0)),
                  pl.BlockSpec((D, Do), lambda i: (0, 0)),
                  pl.BlockSpec((D, Do), lambda i: (0, 0)),
                  pl.BlockSpec((D, Do), lambda i: (0, 0))],
        out_specs=[pl.BlockSpec((PB, Do), lambda i: (i, 0)),
                   pl.BlockSpec((2, Do), lambda i: (0, 0))],
        out_shape=[jax.ShapeDtypeStruct((Mp, Do), jnp.float32),
                   jax.ShapeDtypeStruct((2, Do), jnp.float32)],
        interpret=INTERPRET)(x1, x2, x3, wa, wb, wc)


def _gmax(y6, sc6, bi6, B):
    Mp, Do = y6.shape
    PB = 512
    nbb = (Mp // PB) // B

    def body(y_ref, sc_ref, bi_ref, o_ref):
        i = pl.program_id(0)
        p = y_ref[...] * sc_ref[...] + bi_ref[...]
        z = jnp.where(p >= 0, p, 0.2 * p)
        m = jnp.max(z, axis=0, keepdims=True)

        @pl.when(i % nbb == 0)
        def _():
            o_ref[0] = m

        @pl.when(i % nbb != 0)
        def _():
            o_ref[0] = jnp.maximum(o_ref[0], m)

    return pl.pallas_call(
        body, grid=(Mp // PB,),
        in_specs=[pl.BlockSpec((PB, Do), lambda i: (i, 0)),
                  pl.BlockSpec((1, Do), lambda i: (0, 0)),
                  pl.BlockSpec((1, Do), lambda i: (0, 0))],
        out_specs=pl.BlockSpec((1, 1, Do), lambda i: (i // nbb, 0, 0)),
        out_shape=jax.ShapeDtypeStruct((B, 1, Do), jnp.float32),
        interpret=INTERPRET)(y6, sc6, bi6)


def _mlp8(x1, x2, x3, gmax, wa, wb1, wb2, wb3, B):
    Mp, D = x1.shape
    Do = wa.shape[1]
    PB = 512
    nbb = (Mp // PB) // B

    def body(a_ref, b_ref, c_ref, g_ref, wa_ref, w1_ref, w2_ref, w3_ref,
             y_ref, s_ref):
        i = pl.program_id(0)

        @pl.when(i == 0)
        def _():
            s_ref[...] = jnp.zeros_like(s_ref)

        bias = jnp.dot(g_ref[0], wa_ref[...],
                       preferred_element_type=jnp.float32)
        y = (jnp.dot(a_ref[...], w1_ref[...], preferred_element_type=jnp.float32)
             + jnp.dot(b_ref[...], w2_ref[...], preferred_element_type=jnp.float32)
             + jnp.dot(c_ref[...], w3_ref[...], preferred_element_type=jnp.float32)
             + bias)
        y_ref[...] = y
        s_ref[0:1, :] += jnp.sum(y, axis=0, keepdims=True)
        s_ref[1:2, :] += jnp.sum(y * y, axis=0, keepdims=True)

    return pl.pallas_call(
        body, grid=(Mp // PB,),
        in_specs=[pl.BlockSpec((PB, D), lambda i: (i, 0)),
                  pl.BlockSpec((PB, D), lambda i: (i, 0)),
                  pl.BlockSpec((PB, D), lambda i: (i, 0)),
                  pl.BlockSpec((1, 1, wa.shape[0]), lambda i: (i // nbb, 0, 0)),
                  pl.BlockSpec(wa.shape, lambda i: (0, 0)),
                  pl.BlockSpec((D, Do), lambda i: (0, 0)),
                  pl.BlockSpec((D, Do), lambda i: (0, 0)),
                  pl.BlockSpec((D, Do), lambda i: (0, 0))],
        out_specs=[pl.BlockSpec((PB, Do), lambda i: (i, 0)),
                   pl.BlockSpec((2, Do), lambda i: (0, 0))],
        out_shape=[jax.ShapeDtypeStruct((Mp, Do), jnp.float32),
                   jax.ShapeDtypeStruct((2, Do), jnp.float32)],
        interpret=INTERPRET)(x1, x2, x3, gmax, wa, wb1, wb2, wb3)


def _mlp_mid(y, sc, bi, wt):
    Mp, D = y.shape
    Do = wt.shape[1]
    PB = 512

    def body(y_ref, sc_ref, bi_ref, w_ref, o_ref, s_ref):
        i = pl.program_id(0)

        @pl.when(i == 0)
        def _():
            s_ref[...] = jnp.zeros_like(s_ref)

        p = y_ref[...] * sc_ref[...] + bi_ref[...]
        z = jnp.where(p >= 0, p, 0.2 * p)
        o = jnp.dot(z, w_ref[...], preferred_element_type=jnp.float32)
        o_ref[...] = o
        s_ref[0:1, :] += jnp.sum(o, axis=0, keepdims=True)
        s_ref[1:2, :] += jnp.sum(o * o, axis=0, keepdims=True)

    return pl.pallas_call(
        body, grid=(Mp // PB,),
        in_specs=[pl.BlockSpec((PB, D), lambda i: (i, 0)),
                  pl.BlockSpec((1, D), lambda i: (0, 0)),
                  pl.BlockSpec((1, D), lambda i: (0, 0)),
                  pl.BlockSpec((D, Do), lambda i: (0, 0))],
        out_specs=[pl.BlockSpec((PB, Do), lambda i: (i, 0)),
                   pl.BlockSpec((2, Do), lambda i: (0, 0))],
        out_shape=[jax.ShapeDtypeStruct((Mp, Do), jnp.float32),
                   jax.ShapeDtypeStruct((2, Do), jnp.float32)],
        interpret=INTERPRET)(y, sc, bi, wt)


def _mlp_last(y, sc, bi, wt):
    Mp, D = y.shape
    Do = wt.shape[1]
    PB = 512

    def body(y_ref, sc_ref, bi_ref, w_ref, o_ref):
        p = y_ref[...] * sc_ref[...] + bi_ref[...]
        z = jnp.where(p >= 0, p, 0.2 * p)
        o_ref[...] = jnp.dot(z, w_ref[...], preferred_element_type=jnp.float32)

    return pl.pallas_call(
        body, grid=(Mp // PB,),
        in_specs=[pl.BlockSpec((PB, D), lambda i: (i, 0)),
                  pl.BlockSpec((1, D), lambda i: (0, 0)),
                  pl.BlockSpec((1, D), lambda i: (0, 0)),
                  pl.BlockSpec((D, Do), lambda i: (0, 0))],
        out_specs=pl.BlockSpec((PB, Do), lambda i: (i, 0)),
        out_shape=jax.ShapeDtypeStruct((Mp, Do), jnp.float32),
        interpret=INTERPRET)(y, sc, bi, wt)


def _bnparams(s, cnt, g, b):
    m = s[0] / cnt
    var = s[1] / cnt - m * m
    scl = g / jnp.sqrt(var + EPS)
    return scl.reshape(1, -1), (b - m * scl).reshape(1, -1)


def kernel(x, W1, g1, b1, W2, g2, b2, W3, g3, b3, W4, g4, b4, W5, g5, b5,
           W6, g6, b6, W8, g8, b8, W9, g9, b9, W10, g10, b10, W11):
    B, C0, N = x.shape
    M = B * N

    def edge_block_raw(xt, W, g, b, W2_, g2_, b2_):
        Bb, Nn, C = xt.shape
        Cs = max(C, 8)
        xtp = jnp.pad(xt, ((0, 0), (0, 0), (0, 128 - C)))
        Dm = W.shape[0]
        wf = jnp.zeros((2 * Cs, Dm), x.dtype)
        wf = wf.at[:C].set(W[:, :C].T).at[Cs:Cs + C].set(W[:, C:].T)
        idx = _knn_idx(xtp)
        flat_idx = jnp.transpose(idx[:, :, :KNN_K], (2, 0, 1)).reshape(-1)
        xf = xtp.reshape(M, 128)
        g3_ = _sc_gather(xf, flat_idx).reshape(KNN_K, M, 128)
        mv1 = _edge_stats_raw(g3_, xf, wf, Cs)
        bn1 = jnp.concatenate([mv1, g.reshape(1, -1), b.reshape(1, -1)])
        y3, _ = _edge_conv_raw(g3_, xf, wf, Cs, bn1, W2_.T)
        mv2 = _y_stats(y3)
        bn2 = jnp.concatenate([mv2, g2_.reshape(1, -1), b2_.reshape(1, -1)])
        return _edge_max_y(y3, bn2)

    def edge_block_uv(xt, W, g, b):
        Bb, Nn, C = xt.shape
        A = W[:, :C]
        wu = A.T
        wv = (W[:, C:] - A).T
        xtp = jnp.pad(xt, ((0, 0), (0, 0), (0, 128 - C)))
        wj = jnp.pad(wu, ((0, 128 - C), (0, 128 - wu.shape[1])))
        wvp = jnp.pad(wv, ((0, 128 - C), (0, 0)))
        idx, u, v = _knn_uv(xtp, wj, wvp)
        flat_idx = jnp.transpose(idx[:, :, :KNN_K], (2, 0, 1)).reshape(-1)
        g3_ = _sc_gather(u.reshape(M, 128), flat_idx).reshape(KNN_K, M, 128)
        vf = v.reshape(M, wv.shape[1])
        s1 = _edge_stats(g3_, vf)
        sc1, bi1 = _bnparams(s1, M * KNN_K, g, b)
        return _edge_max_gv(g3_, vf, sc1, bi1)

    xt0 = jnp.transpose(x, (0, 2, 1))
    x1 = edge_block_raw(xt0, W1, g1, b1, W2, g2, b2)
    x2 = edge_block_raw(x1.reshape(B, N, 64), W3, g3, b3, W4, g4, b4)
    x3 = edge_block_uv(x2.reshape(B, N, 64), W5, g5, b5)

    W6t = W6.T
    y6, s6 = _mlp6(x1, x2, x3, W6t[:64], W6t[64:128], W6t[128:192])
    sc6, bi6 = _bnparams(s6, M, g6, b6)
    gm = _gmax(y6, sc6, bi6, B)

    W8t = W8.T
    y8, s8 = _mlp8(x1, x2, x3, gm, W8t[:1024], W8t[1024:1088],
                   W8t[1088:1152], W8t[1152:1216], B)
    sc8, bi8 = _bnparams(s8, M, g8, b8)
    y9, s9 = _mlp_mid(y8, sc8, bi8, W9.T)
    sc9, bi9 = _bnparams(s9, M, g9, b9)
    y10, s10 = _mlp_mid(y9, sc9, bi9, W10.T)
    sc10, bi10 = _bnparams(s10, M, g10, b10)
    out = _mlp_last(y10, sc10, bi10, W11.T)
    return jnp.transpose(out.reshape(B, N, 13), (0, 2, 1))
